# Initial kernel scaffold; baseline (speedup 1.0000x reference)
import jax
import jax.numpy as jnp
from jax.experimental import pallas as pl
from jax.experimental.pallas import tpu as pltpu

_B = 8
_N = 2048
_TQ = 256
_NT = _N // _TQ
_BIG = jnp.float32(1e30)
_HIGH = jax.lax.Precision.HIGHEST
_F32 = jnp.float32


def _first_argmin_onehot(d, col):
    m = jnp.min(d, axis=1, keepdims=True)
    i = jnp.min(jnp.where(d == m, col, _N), axis=1, keepdims=True)
    return m, col == i


def _self_kappa(dist, col, pts_t_full, pts_t_tile, nrm_t_tile):
    _, oh1 = _first_argmin_onehot(dist, col)
    d2 = jnp.where(oh1, _BIG, dist)
    _, oh2 = _first_argmin_onehot(d2, col)
    c1 = jax.lax.dot(oh1.astype(_F32), pts_t_full,
                     precision=_HIGH, preferred_element_type=_F32)
    c2 = jax.lax.dot(oh2.astype(_F32), pts_t_full,
                     precision=_HIGH, preferred_element_type=_F32)

    def term(c):
        v = c - pts_t_tile
        nv = jnp.sqrt(jnp.sum(v * v, axis=1, keepdims=True))
        s = jnp.sum(v * nrm_t_tile, axis=1, keepdims=True)
        return jnp.abs(s / jnp.maximum(nv, 1e-12))

    return 0.5 * (term(c1) + term(c2))


def _curv_kernel(ori_ref, adv_ref, oriT_ref, advT_ref, nrmT_ref,
                 out_ref, kappa_ref, acc_ref):
    b = pl.program_id(0)
    phase = pl.program_id(1)
    t = pl.program_id(2)
    col = jax.lax.broadcasted_iota(jnp.int32, (_TQ, _N), 1)
    row = t * _TQ + jax.lax.broadcasted_iota(jnp.int32, (_TQ, _N), 0)
    sl = pl.ds(t * _TQ, _TQ)

    @pl.when(phase == 0)
    def _():
        p = ori_ref[0]
        pt = oriT_ref[0]
        pt_t = pt[sl, :]
        nt_t = nrmT_ref[0][sl, :]
        n2r = jnp.sum(p * p, axis=0, keepdims=True)
        n2c = jnp.sum(pt_t * pt_t, axis=1, keepdims=True)
        g = jax.lax.dot(pt_t, p, precision=_HIGH, preferred_element_type=_F32)
        dist = (n2c + n2r) - 2.0 * g
        dist = jnp.where(col == row, _BIG, dist)
        kappa_ref[sl, :] = _self_kappa(dist, col, pt, pt_t, nt_t)

    @pl.when(phase == 1)
    def _():
        o = ori_ref[0]
        a = adv_ref[0]
        at_t = advT_ref[0][sl, :]
        nrm_t = nrmT_ref[0]
        n2o = jnp.sum(o * o, axis=0, keepdims=True)
        n2a_c = jnp.sum(at_t * at_t, axis=1, keepdims=True)
        g_ao = jax.lax.dot(at_t, o, precision=_HIGH, preferred_element_type=_F32)
        d_ao = (n2a_c + n2o) - 2.0 * g_ao
        _, oh = _first_argmin_onehot(d_ao, col)
        ohf = oh.astype(_F32)
        nadv = jax.lax.dot(ohf, nrm_t, precision=_HIGH,
                           preferred_element_type=_F32)
        onenn = jax.lax.dot(ohf, kappa_ref[...], precision=_HIGH,
                            preferred_element_type=_F32)
        n2a_r = jnp.sum(a * a, axis=0, keepdims=True)
        g_aa = jax.lax.dot(at_t, a, precision=_HIGH, preferred_element_type=_F32)
        d_aa = (n2a_c + n2a_r) - 2.0 * g_aa
        d_aa = jnp.where(col == row, _BIG, d_aa)
        advk = _self_kappa(d_aa, col, advT_ref[0], at_t, nadv)
        diff = advk - onenn
        part = jnp.reshape(jnp.sum(diff * diff), (1, 1))

        @pl.when(jnp.logical_and(b == 0, t == 0))
        def _():
            acc_ref[...] = jnp.zeros((1, 1), _F32)

        acc_ref[...] += part
        out_ref[...] = acc_ref[...] * (1.0 / (_B * _N))


def kernel(ori_data, adv_data, ori_normal):
    oriT = jnp.transpose(ori_data, (0, 2, 1))
    advT = jnp.transpose(adv_data, (0, 2, 1))
    nrmT = jnp.transpose(ori_normal, (0, 2, 1))
    row_spec = pl.BlockSpec((1, 3, _N), lambda b, p, t: (b, 0, 0))
    t_spec = pl.BlockSpec((1, _N, 3), lambda b, p, t: (b, 0, 0))
    out = pl.pallas_call(
        _curv_kernel,
        grid=(_B, 2, _NT),
        in_specs=[row_spec, row_spec, t_spec, t_spec, t_spec],
        out_specs=pl.BlockSpec((1, 1), lambda b, p, t: (0, 0)),
        out_shape=jax.ShapeDtypeStruct((1, 1), _F32),
        scratch_shapes=[
            pltpu.VMEM((_N, 1), _F32),
            pltpu.VMEM((1, 1), _F32),
        ],
    )(ori_data, adv_data, oriT, advT, nrmT)
    return out[0, 0]


# speedup vs baseline: 19.0623x; 19.0623x over previous
import jax
import jax.numpy as jnp
from jax.experimental import pallas as pl
from jax.experimental.pallas import tpu as pltpu

_B = 8
_N = 2048
_TQ = 256
_NT = _N // _TQ
_BIG = 1e30
_HIGH = jax.lax.Precision.HIGHEST
_F32 = jnp.float32


def _first_argmin_onehot(d, col):
    m = jnp.min(d, axis=1, keepdims=True)
    i = jnp.min(jnp.where(d == m, col, _N), axis=1, keepdims=True)
    return m, col == i


def _self_kappa(dist, col, pts_t_full, pts_t_tile, nrm_t_tile):
    _, oh0 = _first_argmin_onehot(dist, col)
    d1 = jnp.where(oh0, _BIG, dist)
    _, oh1 = _first_argmin_onehot(d1, col)
    d2 = jnp.where(oh1, _BIG, d1)
    _, oh2 = _first_argmin_onehot(d2, col)
    c1 = jax.lax.dot(oh1.astype(_F32), pts_t_full,
                     precision=_HIGH, preferred_element_type=_F32)
    c2 = jax.lax.dot(oh2.astype(_F32), pts_t_full,
                     precision=_HIGH, preferred_element_type=_F32)

    def term(c):
        v = c - pts_t_tile
        nv = jnp.sqrt(jnp.sum(v * v, axis=1, keepdims=True))
        s = jnp.sum(v * nrm_t_tile, axis=1, keepdims=True)
        return jnp.abs(s / jnp.maximum(nv, 1e-12))

    return 0.5 * (term(c1) + term(c2))


def _curv_kernel(ori_ref, adv_ref, oriT_ref, advT_ref, nrmT_ref,
                 out_ref, kappa_ref, acc_ref):
    b = pl.program_id(0)
    phase = pl.program_id(1)
    t = pl.program_id(2)
    col = jax.lax.broadcasted_iota(jnp.int32, (_TQ, _N), 1)
    row = t * _TQ + jax.lax.broadcasted_iota(jnp.int32, (_TQ, _N), 0)
    sl = pl.ds(t * _TQ, _TQ)

    @pl.when(phase == 0)
    def _():
        p = ori_ref[0]
        pt = oriT_ref[0]
        pt_t = oriT_ref[0, sl, :]
        nt_t = nrmT_ref[0, sl, :]
        n2r = jnp.sum(p * p, axis=0, keepdims=True)
        n2c = jnp.sum(pt_t * pt_t, axis=1, keepdims=True)
        g = jax.lax.dot(pt_t, p, preferred_element_type=_F32)
        dist = (n2c + n2r) - 2.0 * g
        kappa_ref[sl, :] = _self_kappa(dist, col, pt, pt_t, nt_t)

    @pl.when(phase == 1)
    def _():
        o = ori_ref[0]
        a = adv_ref[0]
        at_t = advT_ref[0, sl, :]
        nrm_t = nrmT_ref[0]
        n2o = jnp.sum(o * o, axis=0, keepdims=True)
        n2a_c = jnp.sum(at_t * at_t, axis=1, keepdims=True)
        g_ao = jax.lax.dot(at_t, o, preferred_element_type=_F32)
        d_ao = (n2a_c + n2o) - 2.0 * g_ao
        _, oh = _first_argmin_onehot(d_ao, col)
        ohf = oh.astype(_F32)
        nadv = jax.lax.dot(ohf, nrm_t, precision=_HIGH,
                           preferred_element_type=_F32)
        onenn = jax.lax.dot(ohf, kappa_ref[...], precision=_HIGH,
                            preferred_element_type=_F32)
        n2a_r = jnp.sum(a * a, axis=0, keepdims=True)
        g_aa = jax.lax.dot(at_t, a, preferred_element_type=_F32)
        d_aa = (n2a_c + n2a_r) - 2.0 * g_aa
        advk = _self_kappa(d_aa, col, advT_ref[0], at_t, nadv)
        diff = advk - onenn
        part = jnp.reshape(jnp.sum(diff * diff), (1, 1))

        @pl.when(jnp.logical_and(b == 0, t == 0))
        def _():
            acc_ref[...] = jnp.zeros((1, 1), _F32)

        acc_ref[...] += part
        out_ref[...] = acc_ref[...] * (1.0 / (_B * _N))


def kernel(ori_data, adv_data, ori_normal):
    oriT = jnp.transpose(ori_data, (0, 2, 1))
    advT = jnp.transpose(adv_data, (0, 2, 1))
    nrmT = jnp.transpose(ori_normal, (0, 2, 1))
    row_spec = pl.BlockSpec((1, 3, _N), lambda b, p, t: (b, 0, 0))
    t_spec = pl.BlockSpec((1, _N, 3), lambda b, p, t: (b, 0, 0))
    out = pl.pallas_call(
        _curv_kernel,
        grid=(_B, 2, _NT),
        in_specs=[row_spec, row_spec, t_spec, t_spec, t_spec],
        out_specs=pl.BlockSpec((1, 1), lambda b, p, t: (0, 0)),
        out_shape=jax.ShapeDtypeStruct((1, 1), _F32),
        scratch_shapes=[
            pltpu.VMEM((_N, 1), _F32),
            pltpu.VMEM((1, 1), _F32),
        ],
    )(ori_data, adv_data, oriT, advT, nrmT)
    return out[0, 0]


# speedup vs baseline: 22.4568x; 1.1781x over previous
import jax
import jax.numpy as jnp
from jax.experimental import pallas as pl
from jax.experimental.pallas import tpu as pltpu

_B = 8
_N = 2048
_TQ = 256
_NT = _N // _TQ
_BIG = 1e30
_HIGH = jax.lax.Precision.HIGHEST
_F32 = jnp.float32


def _self_kappa(dist, pts_t_full, pts_t_tile, nrm_t_tile):
    m0 = jnp.min(dist, axis=1, keepdims=True)
    d1 = jnp.where(dist == m0, _BIG, dist)
    m1 = jnp.min(d1, axis=1, keepdims=True)
    oh1 = d1 == m1
    d2 = jnp.where(oh1, _BIG, d1)
    m2 = jnp.min(d2, axis=1, keepdims=True)
    oh2 = d2 == m2
    c1 = jax.lax.dot(oh1.astype(_F32), pts_t_full,
                     precision=_HIGH, preferred_element_type=_F32)
    c2 = jax.lax.dot(oh2.astype(_F32), pts_t_full,
                     precision=_HIGH, preferred_element_type=_F32)

    def term(c):
        v = c - pts_t_tile
        nv = jnp.sqrt(jnp.sum(v * v, axis=1, keepdims=True))
        s = jnp.sum(v * nrm_t_tile, axis=1, keepdims=True)
        return jnp.abs(s / jnp.maximum(nv, 1e-12))

    return 0.5 * (term(c1) + term(c2))


def _curv_kernel(ori_ref, adv_ref, oriT_ref, advT_ref, nrmT_ref,
                 out_ref, kappa_ref, acc_ref):
    b = pl.program_id(0)
    phase = pl.program_id(1)
    t = pl.program_id(2)
    sl = pl.ds(t * _TQ, _TQ)

    @pl.when(phase == 0)
    def _():
        p = ori_ref[0]
        pt = oriT_ref[0]
        pt_t = oriT_ref[0, sl, :]
        nt_t = nrmT_ref[0, sl, :]
        n2r = jnp.sum(p * p, axis=0, keepdims=True)
        n2c = jnp.sum(pt_t * pt_t, axis=1, keepdims=True)
        g = jax.lax.dot(pt_t, p, preferred_element_type=_F32)
        dist = (n2c + n2r) - 2.0 * g
        kappa_ref[sl, :] = _self_kappa(dist, pt, pt_t, nt_t)

    @pl.when(phase == 1)
    def _():
        o = ori_ref[0]
        a = adv_ref[0]
        at_t = advT_ref[0, sl, :]
        nrm_t = nrmT_ref[0]
        n2o = jnp.sum(o * o, axis=0, keepdims=True)
        n2a_c = jnp.sum(at_t * at_t, axis=1, keepdims=True)
        g_ao = jax.lax.dot(at_t, o, preferred_element_type=_F32)
        d_ao = (n2a_c + n2o) - 2.0 * g_ao
        m_ao = jnp.min(d_ao, axis=1, keepdims=True)
        ohf = (d_ao == m_ao).astype(_F32)
        nadv = jax.lax.dot(ohf, nrm_t, precision=_HIGH,
                           preferred_element_type=_F32)
        onenn = jax.lax.dot(ohf, kappa_ref[...], precision=_HIGH,
                            preferred_element_type=_F32)
        n2a_r = jnp.sum(a * a, axis=0, keepdims=True)
        g_aa = jax.lax.dot(at_t, a, preferred_element_type=_F32)
        d_aa = (n2a_c + n2a_r) - 2.0 * g_aa
        advk = _self_kappa(d_aa, advT_ref[0], at_t, nadv)
        diff = advk - onenn
        part = jnp.reshape(jnp.sum(diff * diff), (1, 1))

        @pl.when(jnp.logical_and(b == 0, t == 0))
        def _():
            acc_ref[...] = jnp.zeros((1, 1), _F32)

        acc_ref[...] += part
        out_ref[...] = acc_ref[...] * (1.0 / (_B * _N))


def kernel(ori_data, adv_data, ori_normal):
    oriT = jnp.transpose(ori_data, (0, 2, 1))
    advT = jnp.transpose(adv_data, (0, 2, 1))
    nrmT = jnp.transpose(ori_normal, (0, 2, 1))
    row_spec = pl.BlockSpec((1, 3, _N), lambda b, p, t: (b, 0, 0))
    t_spec = pl.BlockSpec((1, _N, 3), lambda b, p, t: (b, 0, 0))
    out = pl.pallas_call(
        _curv_kernel,
        grid=(_B, 2, _NT),
        in_specs=[row_spec, row_spec, t_spec, t_spec, t_spec],
        out_specs=pl.BlockSpec((1, 1), lambda b, p, t: (0, 0)),
        out_shape=jax.ShapeDtypeStruct((1, 1), _F32),
        scratch_shapes=[
            pltpu.VMEM((_N, 1), _F32),
            pltpu.VMEM((1, 1), _F32),
        ],
    )(ori_data, adv_data, oriT, advT, nrmT)
    return out[0, 0]


# speedup vs baseline: 45.5634x; 2.0289x over previous
import jax
import jax.numpy as jnp
from jax.experimental import pallas as pl
from jax.experimental.pallas import tpu as pltpu

_B = 8
_N = 2048
_TQ = 256
_NT = _N // _TQ
_BIG = 1e30
_HIGH = jax.lax.Precision.HIGHEST
_F32 = jnp.float32


def _split3(x):
    hi = x.astype(jnp.bfloat16).astype(_F32)
    r = x - hi
    mid = r.astype(jnp.bfloat16).astype(_F32)
    lo = (r - mid).astype(jnp.bfloat16).astype(_F32)
    return jnp.concatenate([hi, mid, lo], axis=1)


def _self_kappa(dist, pts9, pts_t_tile, nrm_t_tile):
    m0 = jnp.min(dist, axis=1, keepdims=True)
    d1 = jnp.where(dist == m0, _BIG, dist)
    m1 = jnp.min(d1, axis=1, keepdims=True)
    oh1 = d1 == m1
    d2 = jnp.where(oh1, _BIG, d1)
    m2 = jnp.min(d2, axis=1, keepdims=True)
    oh2 = d2 == m2
    r1 = jax.lax.dot(oh1.astype(_F32), pts9, preferred_element_type=_F32)
    r2 = jax.lax.dot(oh2.astype(_F32), pts9, preferred_element_type=_F32)

    def term(r):
        c = (r[:, 0:3] + r[:, 3:6]) + r[:, 6:9]
        v = c - pts_t_tile
        nv = jnp.sqrt(jnp.sum(v * v, axis=1, keepdims=True))
        s = jnp.sum(v * nrm_t_tile, axis=1, keepdims=True)
        return jnp.abs(s / jnp.maximum(nv, 1e-12))

    return 0.5 * (term(r1) + term(r2))


def _curv_kernel(ori_ref, adv_ref, oriT_ref, advT_ref, nrmT_ref,
                 out_ref, kappa_ref, acc_ref):
    b = pl.program_id(0)
    phase = pl.program_id(1)
    t = pl.program_id(2)
    sl = pl.ds(t * _TQ, _TQ)

    @pl.when(phase == 0)
    def _():
        p = ori_ref[0]
        pt = oriT_ref[0]
        pt_t = oriT_ref[0, sl, :]
        nt_t = nrmT_ref[0, sl, :]
        n2r = jnp.sum(p * p, axis=0, keepdims=True)
        n2c = jnp.sum(pt_t * pt_t, axis=1, keepdims=True)
        g = jax.lax.dot(pt_t, p, preferred_element_type=_F32)
        dist = (n2c + n2r) - 2.0 * g
        kappa_ref[sl, :] = _self_kappa(dist, _split3(pt), pt_t, nt_t)

    @pl.when(phase == 1)
    def _():
        o = ori_ref[0]
        a = adv_ref[0]
        at_t = advT_ref[0, sl, :]
        nrm_t = nrmT_ref[0]
        n2o = jnp.sum(o * o, axis=0, keepdims=True)
        n2a_c = jnp.sum(at_t * at_t, axis=1, keepdims=True)
        g_ao = jax.lax.dot(at_t, o, preferred_element_type=_F32)
        d_ao = (n2a_c + n2o) - 2.0 * g_ao
        m_ao = jnp.min(d_ao, axis=1, keepdims=True)
        ohf = (d_ao == m_ao).astype(_F32)
        payload = jnp.concatenate([_split3(nrm_t), _split3(kappa_ref[...])],
                                  axis=1)
        raw = jax.lax.dot(ohf, payload, preferred_element_type=_F32)
        nadv = (raw[:, 0:3] + raw[:, 3:6]) + raw[:, 6:9]
        onenn = (raw[:, 9:10] + raw[:, 10:11]) + raw[:, 11:12]
        n2a_r = jnp.sum(a * a, axis=0, keepdims=True)
        g_aa = jax.lax.dot(at_t, a, preferred_element_type=_F32)
        d_aa = (n2a_c + n2a_r) - 2.0 * g_aa
        advk = _self_kappa(d_aa, _split3(advT_ref[0]), at_t, nadv)
        diff = advk - onenn
        part = jnp.reshape(jnp.sum(diff * diff), (1, 1))

        @pl.when(jnp.logical_and(b == 0, t == 0))
        def _():
            acc_ref[...] = jnp.zeros((1, 1), _F32)

        acc_ref[...] += part
        out_ref[...] = acc_ref[...] * (1.0 / (_B * _N))


def kernel(ori_data, adv_data, ori_normal):
    oriT = jnp.transpose(ori_data, (0, 2, 1))
    advT = jnp.transpose(adv_data, (0, 2, 1))
    nrmT = jnp.transpose(ori_normal, (0, 2, 1))
    row_spec = pl.BlockSpec((1, 3, _N), lambda b, p, t: (b, 0, 0))
    t_spec = pl.BlockSpec((1, _N, 3), lambda b, p, t: (b, 0, 0))
    out = pl.pallas_call(
        _curv_kernel,
        grid=(_B, 2, _NT),
        in_specs=[row_spec, row_spec, t_spec, t_spec, t_spec],
        out_specs=pl.BlockSpec((1, 1), lambda b, p, t: (0, 0)),
        out_shape=jax.ShapeDtypeStruct((1, 1), _F32),
        scratch_shapes=[
            pltpu.VMEM((_N, 1), _F32),
            pltpu.VMEM((1, 1), _F32),
        ],
    )(ori_data, adv_data, oriT, advT, nrmT)
    return out[0, 0]


# speedup vs baseline: 64.5817x; 1.4174x over previous
import jax
import jax.numpy as jnp
from jax.experimental import pallas as pl
from jax.experimental.pallas import tpu as pltpu

_B = 8
_N = 2048
_TQ = 256
_NT = _N // _TQ
_LANES = 128
_NSL = _N // _LANES
_BIG = 1e30
_F32 = jnp.float32
_BF16 = jnp.bfloat16


def _split3(x):
    hi = x.astype(_BF16).astype(_F32)
    r = x - hi
    mid = r.astype(_BF16).astype(_F32)
    lo = (r - mid).astype(_BF16).astype(_F32)
    return jnp.concatenate([hi, mid, lo], axis=1).astype(_BF16)


def _lane_top2(dist):
    m0 = dist[:, 0:_LANES]
    m1 = jnp.full_like(m0, _BIG)
    for k in range(1, _NSL):
        x = dist[:, k * _LANES:(k + 1) * _LANES]
        lo = jnp.minimum(m0, x)
        hi = jnp.maximum(m0, x)
        m1 = jnp.minimum(m1, hi)
        m0 = lo
    return m0, m1


def _top3_vals(dist):
    m0, m1 = _lane_top2(dist)
    mm0 = jnp.min(m0, axis=1, keepdims=True)
    a0 = jnp.where(m0 == mm0, _BIG, m0)
    a1 = jnp.where(m1 == mm0, _BIG, m1)
    mm1 = jnp.min(jnp.minimum(a0, a1), axis=1, keepdims=True)
    b0 = jnp.where(a0 == mm1, _BIG, a0)
    b1 = jnp.where(a1 == mm1, _BIG, a1)
    mm2 = jnp.min(jnp.minimum(b0, b1), axis=1, keepdims=True)
    return mm1, mm2


def _self_kappa(dist, pts9, pts_t_tile, nrm_t_tile):
    mm1, mm2 = _top3_vals(dist)
    oh1 = (dist == mm1).astype(_BF16)
    oh2 = (dist == mm2).astype(_BF16)
    r1 = jax.lax.dot(oh1, pts9, preferred_element_type=_F32)
    r2 = jax.lax.dot(oh2, pts9, preferred_element_type=_F32)

    def term(r):
        c = (r[:, 0:3] + r[:, 3:6]) + r[:, 6:9]
        v = c - pts_t_tile
        nv = jnp.sqrt(jnp.sum(v * v, axis=1, keepdims=True))
        s = jnp.sum(v * nrm_t_tile, axis=1, keepdims=True)
        return jnp.abs(s / jnp.maximum(nv, 1e-12))

    return 0.5 * (term(r1) + term(r2))


def _curv_kernel(ori_ref, adv_ref, oriT_ref, advT_ref, nrmT_ref,
                 out_ref, kappa_ref, acc_ref, ori9_ref, adv9_ref, pay12_ref):
    b = pl.program_id(0)
    phase = pl.program_id(1)
    t = pl.program_id(2)
    sl = pl.ds(t * _TQ, _TQ)

    @pl.when(jnp.logical_and(phase == 0, t == 0))
    def _():
        ori9_ref[...] = _split3(oriT_ref[0])
        adv9_ref[...] = _split3(advT_ref[0])

    @pl.when(phase == 0)
    def _():
        p = ori_ref[0]
        pt_t = oriT_ref[0, sl, :]
        nt_t = nrmT_ref[0, sl, :]
        n2r = jnp.sum(p * p, axis=0, keepdims=True)
        n2c = jnp.sum(pt_t * pt_t, axis=1, keepdims=True)
        g2 = jax.lax.dot(pt_t * -2.0, p, preferred_element_type=_F32)
        dist = (n2c + n2r) + g2
        kappa_ref[sl, :] = _self_kappa(dist, ori9_ref[...], pt_t, nt_t)

    @pl.when(jnp.logical_and(phase == 1, t == 0))
    def _():
        pay12_ref[...] = jnp.concatenate(
            [_split3(nrmT_ref[0]), _split3(kappa_ref[...])], axis=1)

    @pl.when(phase == 1)
    def _():
        o = ori_ref[0]
        a = adv_ref[0]
        at_t = advT_ref[0, sl, :]
        n2o = jnp.sum(o * o, axis=0, keepdims=True)
        n2a_c = jnp.sum(at_t * at_t, axis=1, keepdims=True)
        at2 = at_t * -2.0
        g_ao = jax.lax.dot(at2, o, preferred_element_type=_F32)
        d_ao = (n2a_c + n2o) + g_ao
        cm = d_ao[:, 0:_LANES]
        for k in range(1, _NSL):
            cm = jnp.minimum(cm, d_ao[:, k * _LANES:(k + 1) * _LANES])
        mm = jnp.min(cm, axis=1, keepdims=True)
        ohf = (d_ao == mm).astype(_BF16)
        raw = jax.lax.dot(ohf, pay12_ref[...], preferred_element_type=_F32)
        nadv = (raw[:, 0:3] + raw[:, 3:6]) + raw[:, 6:9]
        onenn = (raw[:, 9:10] + raw[:, 10:11]) + raw[:, 11:12]
        n2a_r = jnp.sum(a * a, axis=0, keepdims=True)
        g_aa = jax.lax.dot(at2, a, preferred_element_type=_F32)
        d_aa = (n2a_c + n2a_r) + g_aa
        advk = _self_kappa(d_aa, adv9_ref[...], at_t, nadv)
        diff = advk - onenn
        part = jnp.reshape(jnp.sum(diff * diff), (1, 1))

        @pl.when(jnp.logical_and(b == 0, t == 0))
        def _():
            acc_ref[...] = jnp.zeros((1, 1), _F32)

        acc_ref[...] += part
        out_ref[...] = acc_ref[...] * (1.0 / (_B * _N))


def kernel(ori_data, adv_data, ori_normal):
    oriT = jnp.transpose(ori_data, (0, 2, 1))
    advT = jnp.transpose(adv_data, (0, 2, 1))
    nrmT = jnp.transpose(ori_normal, (0, 2, 1))
    row_spec = pl.BlockSpec((1, 3, _N), lambda b, p, t: (b, 0, 0))
    t_spec = pl.BlockSpec((1, _N, 3), lambda b, p, t: (b, 0, 0))
    out = pl.pallas_call(
        _curv_kernel,
        grid=(_B, 2, _NT),
        in_specs=[row_spec, row_spec, t_spec, t_spec, t_spec],
        out_specs=pl.BlockSpec((1, 1), lambda b, p, t: (0, 0)),
        out_shape=jax.ShapeDtypeStruct((1, 1), _F32),
        scratch_shapes=[
            pltpu.VMEM((_N, 1), _F32),
            pltpu.VMEM((1, 1), _F32),
            pltpu.VMEM((_N, 9), _BF16),
            pltpu.VMEM((_N, 9), _BF16),
            pltpu.VMEM((_N, 12), _BF16),
        ],
    )(ori_data, adv_data, oriT, advT, nrmT)
    return out[0, 0]
